# Initial kernel scaffold; baseline (speedup 1.0000x reference)
import jax
import jax.numpy as jnp
from jax.experimental import pallas as pl


def kernel(xy, points, affine_alpha, affine_beta):
    raise NotImplementedError("write your pallas kernel here")


# speedup vs baseline: 2.8064x; 2.8064x over previous
import functools

import jax
import jax.numpy as jnp
from jax import lax
from jax.experimental import pallas as pl
from jax.experimental.pallas import tpu as pltpu
from jax.experimental.pallas import tpu_sc as plsc

_B, _N, _D = 8, 4096, 128
_S = 1024
_K = 32
_C2 = _D + 2
_COUT = 2 * _D + 2
_SR = _S // 128
_NR = _N // 128


def _fps_body(xy_ref, sidx_ref, nx_ref, ny_ref):
    xv = xy_ref[:, 0]
    yv = xy_ref[:, 1]
    iota_n = (lax.broadcasted_iota(jnp.int32, (1, _NR, 128), 1) * 128
              + lax.broadcasted_iota(jnp.int32, (1, _NR, 128), 2))
    iota_s = (lax.broadcasted_iota(jnp.int32, (1, _SR, 128), 1) * 128
              + lax.broadcasted_iota(jnp.int32, (1, _SR, 128), 2))

    def body(i, st):
        dist, f, sacc, cxa, cya = st
        m = iota_n == f
        cx = jnp.sum(jnp.where(m, xv, 0.0), axis=(1, 2), keepdims=True)
        cy = jnp.sum(jnp.where(m, yv, 0.0), axis=(1, 2), keepdims=True)
        pm = iota_s == i
        sacc = jnp.where(pm, f, sacc)
        cxa = jnp.where(pm, cx, cxa)
        cya = jnp.where(pm, cy, cya)
        d = (xv - cx) ** 2 + (yv - cy) ** 2
        dist = jnp.minimum(dist, d)
        gmax = jnp.max(dist, axis=(1, 2), keepdims=True)
        f = jnp.min(jnp.where(dist == gmax, iota_n, 2 ** 30),
                    axis=(1, 2), keepdims=True)
        return dist, f, sacc, cxa, cya

    dist0 = jnp.full((_B, _NR, 128), 1e10, dtype=jnp.float32)
    f0 = jnp.zeros((_B, 1, 1), dtype=jnp.int32)
    sacc0 = jnp.zeros((_B, _SR, 128), dtype=jnp.int32)
    z0 = jnp.zeros((_B, _SR, 128), dtype=jnp.float32)
    _, _, sacc, cxa, cya = lax.fori_loop(0, _S, body,
                                         (dist0, f0, sacc0, z0, z0))
    sidx_ref[...] = sacc
    nx_ref[...] = cxa
    ny_ref[...] = cya


def _fps(xy):
    xyt = xy.transpose(0, 2, 1).reshape(_B, 2, _NR, 128)
    sidx, nx, ny = pl.pallas_call(
        _fps_body,
        out_shape=(
            jax.ShapeDtypeStruct((_B, _SR, 128), jnp.int32),
            jax.ShapeDtypeStruct((_B, _SR, 128), jnp.float32),
            jax.ShapeDtypeStruct((_B, _SR, 128), jnp.float32),
        ),
    )(xyt)
    return sidx, nx, ny


_RT = 8


def _knn_body(xyt_ref, nqx_ref, nqy_ref, idx_ref):
    b = pl.program_id(0)
    x = xyt_ref[0, 0]
    y = xyt_ref[0, 1]
    sx = nqx_ref[0]
    sy = nqy_ref[0]
    q = jnp.concatenate([sx, sy], axis=1)
    xyt2 = jnp.concatenate([x, y], axis=0)
    t = jax.lax.dot_general(q, xyt2, (((1,), (0,)), ((), ())),
                            preferred_element_type=jnp.float32)
    d = -2.0 * t + (sx * sx + sy * sy) + (x * x + y * y)
    iota = lax.broadcasted_iota(jnp.int32, (_RT, _N), 1)
    cols = []
    for _ in range(_K):
        gmin = jnp.min(d, axis=1, keepdims=True)
        nidx = jnp.min(jnp.where(d == gmin, iota, 2 ** 30),
                       axis=1, keepdims=True)
        cols.append(nidx)
        d = jnp.where(iota == nidx, jnp.inf, d)
    idx_ref[0] = jnp.concatenate(cols, axis=1) + b * _N


def _knn(xy, nx, ny):
    xyt = xy.transpose(0, 2, 1).reshape(_B, 2, 1, _N)
    nqx = nx.reshape(_B, _S, 1)
    nqy = ny.reshape(_B, _S, 1)
    idx = pl.pallas_call(
        _knn_body,
        grid=(_B, _S // _RT),
        in_specs=[
            pl.BlockSpec((1, 2, 1, _N), lambda b, j: (b, 0, 0, 0)),
            pl.BlockSpec((1, _RT, 1), lambda b, j: (b, j, 0)),
            pl.BlockSpec((1, _RT, 1), lambda b, j: (b, j, 0)),
        ],
        out_specs=pl.BlockSpec((1, _RT, _K), lambda b, j: (b, j, 0)),
        out_shape=jax.ShapeDtypeStruct((_B, _S, _K), jnp.int32),
    )(xyt, nqx, nqy)
    return idx


_NC, _NS = 2, 16
_NW = _NC * _NS
_GCH = 256
_GPW = (_B * _S * _K) // _NW
_SPW = (_B * _S) // _NW


def _gather_body(pts_hbm, xyp_hbm, gidx_hbm, sgidx_hbm,
                 gpts_hbm, gxy_hbm, spts_hbm,
                 idx_v, rows_v, xyrows_v, sidx_v, srows_v, sem):
    wid = lax.axis_index("s") * _NC + lax.axis_index("c")
    base0 = wid * _GPW

    def chunk(i, carry):
        base = pl.multiple_of(base0 + i * _GCH, _GCH)
        pltpu.sync_copy(gidx_hbm.at[pl.ds(base, _GCH)], idx_v)
        pltpu.async_copy(pts_hbm.at[idx_v], rows_v, sem).wait()
        pltpu.sync_copy(rows_v, gpts_hbm.at[pl.ds(base, _GCH)])
        pltpu.async_copy(xyp_hbm.at[idx_v], xyrows_v, sem).wait()
        pltpu.sync_copy(xyrows_v, gxy_hbm.at[pl.ds(base, _GCH)])
        return carry

    lax.fori_loop(0, _GPW // _GCH, chunk, 0)

    sbase = pl.multiple_of(wid * _SPW, _SPW)
    pltpu.sync_copy(sgidx_hbm.at[pl.ds(sbase, _SPW)], sidx_v)
    pltpu.async_copy(pts_hbm.at[sidx_v], srows_v, sem).wait()
    pltpu.sync_copy(srows_v, spts_hbm.at[pl.ds(sbase, _SPW)])


def _gather(points, xy, gidx, sgidx):
    pts_tab = points.reshape(_B * _N, _D)
    xy_tab = jnp.pad(xy.reshape(_B * _N, 2), ((0, 0), (0, 14)))
    mesh = plsc.VectorSubcoreMesh(core_axis_name="c", subcore_axis_name="s")
    run = pl.kernel(
        _gather_body,
        mesh=mesh,
        compiler_params=pltpu.CompilerParams(use_tc_tiling_on_sc=False),
        out_type=(
            jax.ShapeDtypeStruct((_B * _S * _K, _D), jnp.float32),
            jax.ShapeDtypeStruct((_B * _S * _K, 16), jnp.float32),
            jax.ShapeDtypeStruct((_B * _S, _D), jnp.float32),
        ),
        scratch_types=[
            pltpu.VMEM((_GCH,), jnp.int32),
            pltpu.VMEM((_GCH, _D), jnp.float32),
            pltpu.VMEM((_GCH, 16), jnp.float32),
            pltpu.VMEM((_SPW,), jnp.int32),
            pltpu.VMEM((_SPW, _D), jnp.float32),
            pltpu.SemaphoreType.DMA,
        ],
    )
    return run(pts_tab, xy_tab, gidx.reshape(-1), sgidx.reshape(-1))


_TS1 = 128
_M_TOT = _S * _K * _C2


def _stats_body(g1_ref, g2_ref, st_ref):
    j = pl.program_id(1)
    g1 = g1_ref[...]
    m1 = jnp.mean(g1, axis=2, keepdims=True)
    y1 = g1 - m1
    g2 = g2_ref[0, :, :, 0:2]
    m2 = jnp.mean(g2, axis=1, keepdims=True)
    y2 = g2 - m2
    s = jnp.sum(y1) + jnp.sum(y2)
    q = jnp.sum(y1 * y1) + jnp.sum(y2 * y2)
    lane = lax.broadcasted_iota(jnp.int32, (1, 8, 128), 2)
    sub = lax.broadcasted_iota(jnp.int32, (1, 8, 128), 1)
    oh0 = jnp.where((lane == 0) & (sub == 0), 1.0, 0.0)
    oh1 = jnp.where((lane == 1) & (sub == 0), 1.0, 0.0)
    oh2 = (lane == 2) & (sub == 0)

    @pl.when(j == 0)
    def _():
        st_ref[...] = jnp.zeros((1, 8, 128), jnp.float32)

    st_ref[...] += s * oh0 + q * oh1

    @pl.when(j == (_S // _TS1) - 1)
    def _():
        st = st_ref[...]
        s_tot = jnp.sum(st * oh0)
        q_tot = jnp.sum(st * oh1)
        var = (q_tot - s_tot * s_tot / _M_TOT) / (_M_TOT - 1)
        std = jnp.sqrt(var)
        st_ref[...] = jnp.where(oh2, std, st)


def _stats(g1, g2):
    return pl.pallas_call(
        _stats_body,
        grid=(_B, _S // _TS1),
        in_specs=[
            pl.BlockSpec((1, _TS1, _K, _D), lambda b, j: (b, j, 0, 0)),
            pl.BlockSpec((1, _TS1, _K, 16), lambda b, j: (b, j, 0, 0)),
        ],
        out_specs=pl.BlockSpec((1, 8, 128), lambda b, j: (b, 0, 0)),
        out_shape=jax.ShapeDtypeStruct((_B, 8, 128), jnp.float32),
    )(g1, g2)


_TS2 = 64


def _final_body(g1_ref, g2_ref, sp_ref, st_ref, a_ref, b_ref, out_ref):
    std = st_ref[0, 0, 2]
    denom = std + 1e-5
    a1 = a_ref[0, 0, :, 0:_D]
    b1 = b_ref[0, 0, :, 0:_D]
    a2 = a_ref[0, 0, :, _D:_C2]
    b2 = b_ref[0, 0, :, _D:_C2]
    g1 = g1_ref[...]
    m1 = jnp.mean(g1, axis=2, keepdims=True)
    n1 = a1[None, None] * ((g1 - m1) / denom) + b1[None, None]
    g2 = g2_ref[0, :, :, 0:2]
    m2 = jnp.mean(g2, axis=1, keepdims=True)
    n2 = a2[None, None] * ((g2 - m2) / denom)[None] + b2[None, None]
    rep = jnp.broadcast_to(sp_ref[0][:, None, :], (_TS2, _K, _D))[None]
    out_ref[...] = jnp.concatenate([n1, n2, rep], axis=-1)


def _final(g1, g2, spts, st, alpha, beta):
    a4 = alpha.reshape(1, 1, 1, _C2)
    b4 = beta.reshape(1, 1, 1, _C2)
    return pl.pallas_call(
        _final_body,
        grid=(_B, _S // _TS2),
        in_specs=[
            pl.BlockSpec((1, _TS2, _K, _D), lambda b, j: (b, j, 0, 0)),
            pl.BlockSpec((1, _TS2, _K, 16), lambda b, j: (b, j, 0, 0)),
            pl.BlockSpec((1, _TS2, _D), lambda b, j: (b, j, 0)),
            pl.BlockSpec((1, 8, 128), lambda b, j: (b, 0, 0)),
            pl.BlockSpec((1, 1, 1, _C2), lambda b, j: (0, 0, 0, 0)),
            pl.BlockSpec((1, 1, 1, _C2), lambda b, j: (0, 0, 0, 0)),
        ],
        out_specs=pl.BlockSpec((1, _TS2, _K, _COUT), lambda b, j: (b, j, 0, 0)),
        out_shape=jax.ShapeDtypeStruct((_B, _S, _K, _COUT), jnp.float32),
    )(g1, g2, spts, st, a4, b4)


def kernel(xy, points, affine_alpha, affine_beta):
    sidx, nx, ny = _fps(xy)
    new_xy = jnp.stack([nx.reshape(_B, _S), ny.reshape(_B, _S)], axis=-1)
    idx = _knn(xy, nx, ny)
    boff = (jnp.arange(_B, dtype=jnp.int32) * _N).reshape(_B, 1, 1)
    sgidx = sidx.reshape(_B, _S) + boff[:, :, 0]
    gpts, gxy, spts = _gather(points, xy, idx, sgidx)
    g1 = gpts.reshape(_B, _S, _K, _D)
    g2 = gxy.reshape(_B, _S, _K, 16)
    sp = spts.reshape(_B, _S, _D)
    st = _stats(g1, g2)
    new_points = _final(g1, g2, sp, st, affine_alpha, affine_beta)
    return new_xy, new_points


# speedup vs baseline: 4.1093x; 1.4643x over previous
import functools

import jax
import jax.numpy as jnp
from jax import lax
from jax.experimental import pallas as pl
from jax.experimental.pallas import tpu as pltpu
from jax.experimental.pallas import tpu_sc as plsc

_B, _N, _D = 8, 4096, 128
_S = 1024
_K = 32
_C2 = _D + 2
_COUT = 2 * _D + 2
_SR = _S // 128
_NR = _N // 128


def _fps_body(xy_ref, sidx_ref, nx_ref, ny_ref):
    xv = xy_ref[:, 0]
    yv = xy_ref[:, 1]
    iota_n = (lax.broadcasted_iota(jnp.int32, (1, _NR, 128), 1) * 128
              + lax.broadcasted_iota(jnp.int32, (1, _NR, 128), 2))
    iota_s = (lax.broadcasted_iota(jnp.int32, (1, _SR, 128), 1) * 128
              + lax.broadcasted_iota(jnp.int32, (1, _SR, 128), 2))

    def body(i, st):
        dist, f, sacc, cxa, cya = st
        m = iota_n == f
        cx = jnp.sum(jnp.where(m, xv, 0.0), axis=(1, 2), keepdims=True)
        cy = jnp.sum(jnp.where(m, yv, 0.0), axis=(1, 2), keepdims=True)
        pm = iota_s == i
        sacc = jnp.where(pm, f, sacc)
        cxa = jnp.where(pm, cx, cxa)
        cya = jnp.where(pm, cy, cya)
        d = (xv - cx) ** 2 + (yv - cy) ** 2
        dist = jnp.minimum(dist, d)
        gmax = jnp.max(dist, axis=(1, 2), keepdims=True)
        f = jnp.min(jnp.where(dist == gmax, iota_n, 2 ** 30),
                    axis=(1, 2), keepdims=True)
        return dist, f, sacc, cxa, cya

    dist0 = jnp.full((_B, _NR, 128), 1e10, dtype=jnp.float32)
    f0 = jnp.zeros((_B, 1, 1), dtype=jnp.int32)
    sacc0 = jnp.zeros((_B, _SR, 128), dtype=jnp.int32)
    z0 = jnp.zeros((_B, _SR, 128), dtype=jnp.float32)
    _, _, sacc, cxa, cya = lax.fori_loop(0, _S, body,
                                         (dist0, f0, sacc0, z0, z0))
    sidx_ref[...] = sacc
    nx_ref[...] = cxa
    ny_ref[...] = cya


def _fps(xy):
    xyt = xy.transpose(0, 2, 1).reshape(_B, 2, _NR, 128)
    sidx, nx, ny = pl.pallas_call(
        _fps_body,
        out_shape=(
            jax.ShapeDtypeStruct((_B, _SR, 128), jnp.int32),
            jax.ShapeDtypeStruct((_B, _SR, 128), jnp.float32),
            jax.ShapeDtypeStruct((_B, _SR, 128), jnp.float32),
        ),
    )(xyt)
    return sidx, nx, ny


_RT = 16
_DEPTH = 6


def _knn_body(xyt_ref, nqx_ref, nqy_ref, idx_ref):
    b = pl.program_id(0)
    x = xyt_ref[0, 0]
    y = xyt_ref[0, 1]
    sx = nqx_ref[0]
    sy = nqy_ref[0]
    q = jnp.concatenate([sx, sy], axis=1)
    xyt2 = jnp.concatenate([x, y], axis=0)
    t = jax.lax.dot_general(q, xyt2, (((1,), (0,)), ((), ())),
                            preferred_element_type=jnp.float32)
    d0 = -2.0 * t + (sx * sx + sy * sy) + (x * x + y * y)

    big = jnp.int32(2 ** 30)
    inf = jnp.float32(jnp.inf)
    lane = lax.broadcasted_iota(jnp.int32, (_RT, 128), 1)
    n3 = (lax.broadcasted_iota(jnp.int32, (1, _NR, 128), 1) * 128
          + lax.broadcasted_iota(jnp.int32, (1, _NR, 128), 2))

    rem = d0.reshape(_RT, _NR, 128)
    sv, sn = [], []
    for lev in range(_DEPTH):
        mv = rem[:, 0]
        mn = lane
        for j in range(1, _NR):
            c = rem[:, j]
            lt = c < mv
            mn = jnp.where(lt, lane + j * 128, mn)
            mv = jnp.where(lt, c, mv)
        sv.append(mv)
        sn.append(mn)
        if lev < _DEPTH - 1:
            rem = jnp.where(n3 == mn[:, None, :], inf, rem)

    cnt = jnp.zeros((_RT, 128), jnp.int32)
    hv, hn = sv[0], sn[0]
    cols = []
    for _ in range(_K):
        gmin = jnp.min(hv, axis=1, keepdims=True)
        nsel = jnp.min(jnp.where(hv == gmin, hn, big), axis=1, keepdims=True)
        cols.append(nsel)
        consumed = hn == nsel
        cnt = cnt + jnp.where(consumed, 1, 0)
        nv = jnp.full((_RT, 128), inf, jnp.float32)
        nn = jnp.full((_RT, 128), big, jnp.int32)
        for lev in range(1, _DEPTH):
            m = cnt == lev
            nv = jnp.where(m, sv[lev], nv)
            nn = jnp.where(m, sn[lev], nn)
        hv = jnp.where(consumed, nv, hv)
        hn = jnp.where(consumed, nn, hn)
    fast = jnp.concatenate(cols, axis=1)

    def _naive():
        d = d0
        iota = lax.broadcasted_iota(jnp.int32, (_RT, _N), 1)
        cs = []
        for _ in range(_K):
            g = jnp.min(d, axis=1, keepdims=True)
            ni = jnp.min(jnp.where(d == g, iota, big), axis=1, keepdims=True)
            cs.append(ni)
            d = jnp.where(iota == ni, inf, d)
        return jnp.concatenate(cs, axis=1)

    res = lax.cond(jnp.max(cnt) >= _DEPTH, _naive, lambda: fast)
    idx_ref[0] = res + b * _N


def _knn(xy, nx, ny):
    xyt = xy.transpose(0, 2, 1).reshape(_B, 2, 1, _N)
    nqx = nx.reshape(_B, _S, 1)
    nqy = ny.reshape(_B, _S, 1)
    idx = pl.pallas_call(
        _knn_body,
        grid=(_B, _S // _RT),
        in_specs=[
            pl.BlockSpec((1, 2, 1, _N), lambda b, j: (b, 0, 0, 0)),
            pl.BlockSpec((1, _RT, 1), lambda b, j: (b, j, 0)),
            pl.BlockSpec((1, _RT, 1), lambda b, j: (b, j, 0)),
        ],
        out_specs=pl.BlockSpec((1, _RT, _K), lambda b, j: (b, j, 0)),
        out_shape=jax.ShapeDtypeStruct((_B, _S, _K), jnp.int32),
    )(xyt, nqx, nqy)
    return idx


_NC, _NS = 2, 16
_NW = _NC * _NS
_GCH = 256
_GPW = (_B * _S * _K) // _NW
_SPW = (_B * _S) // _NW


def _gather_body(pts_hbm, xyp_hbm, gidx_hbm, sgidx_hbm,
                 gpts_hbm, gxy_hbm, spts_hbm,
                 idx_v, rows_v, xyrows_v, sidx_v, srows_v, sem):
    wid = lax.axis_index("s") * _NC + lax.axis_index("c")
    base0 = wid * _GPW

    def chunk(i, carry):
        base = pl.multiple_of(base0 + i * _GCH, _GCH)
        pltpu.sync_copy(gidx_hbm.at[pl.ds(base, _GCH)], idx_v)
        pltpu.async_copy(pts_hbm.at[idx_v], rows_v, sem).wait()
        pltpu.sync_copy(rows_v, gpts_hbm.at[pl.ds(base, _GCH)])
        pltpu.async_copy(xyp_hbm.at[idx_v], xyrows_v, sem).wait()
        pltpu.sync_copy(xyrows_v, gxy_hbm.at[pl.ds(base, _GCH)])
        return carry

    lax.fori_loop(0, _GPW // _GCH, chunk, 0)

    sbase = pl.multiple_of(wid * _SPW, _SPW)
    pltpu.sync_copy(sgidx_hbm.at[pl.ds(sbase, _SPW)], sidx_v)
    pltpu.async_copy(pts_hbm.at[sidx_v], srows_v, sem).wait()
    pltpu.sync_copy(srows_v, spts_hbm.at[pl.ds(sbase, _SPW)])


def _gather(points, xy, gidx, sgidx):
    pts_tab = points.reshape(_B * _N, _D)
    xy_tab = jnp.pad(xy.reshape(_B * _N, 2), ((0, 0), (0, 14)))
    mesh = plsc.VectorSubcoreMesh(core_axis_name="c", subcore_axis_name="s")
    run = pl.kernel(
        _gather_body,
        mesh=mesh,
        compiler_params=pltpu.CompilerParams(use_tc_tiling_on_sc=False),
        out_type=(
            jax.ShapeDtypeStruct((_B * _S * _K, _D), jnp.float32),
            jax.ShapeDtypeStruct((_B * _S * _K, 16), jnp.float32),
            jax.ShapeDtypeStruct((_B * _S, _D), jnp.float32),
        ),
        scratch_types=[
            pltpu.VMEM((_GCH,), jnp.int32),
            pltpu.VMEM((_GCH, _D), jnp.float32),
            pltpu.VMEM((_GCH, 16), jnp.float32),
            pltpu.VMEM((_SPW,), jnp.int32),
            pltpu.VMEM((_SPW, _D), jnp.float32),
            pltpu.SemaphoreType.DMA,
        ],
    )
    return run(pts_tab, xy_tab, gidx.reshape(-1), sgidx.reshape(-1))


_TS1 = 128
_M_TOT = _S * _K * _C2


def _stats_body(g1_ref, g2_ref, st_ref):
    j = pl.program_id(1)
    g1 = g1_ref[...]
    m1 = jnp.mean(g1, axis=2, keepdims=True)
    y1 = g1 - m1
    g2 = g2_ref[0, :, :, 0:2]
    m2 = jnp.mean(g2, axis=1, keepdims=True)
    y2 = g2 - m2
    s = jnp.sum(y1) + jnp.sum(y2)
    q = jnp.sum(y1 * y1) + jnp.sum(y2 * y2)
    lane = lax.broadcasted_iota(jnp.int32, (1, 8, 128), 2)
    sub = lax.broadcasted_iota(jnp.int32, (1, 8, 128), 1)
    oh0 = jnp.where((lane == 0) & (sub == 0), 1.0, 0.0)
    oh1 = jnp.where((lane == 1) & (sub == 0), 1.0, 0.0)
    oh2 = (lane == 2) & (sub == 0)

    @pl.when(j == 0)
    def _():
        st_ref[...] = jnp.zeros((1, 8, 128), jnp.float32)

    st_ref[...] += s * oh0 + q * oh1

    @pl.when(j == (_S // _TS1) - 1)
    def _():
        st = st_ref[...]
        s_tot = jnp.sum(st * oh0)
        q_tot = jnp.sum(st * oh1)
        var = (q_tot - s_tot * s_tot / _M_TOT) / (_M_TOT - 1)
        std = jnp.sqrt(var)
        st_ref[...] = jnp.where(oh2, std, st)


def _stats(g1, g2):
    return pl.pallas_call(
        _stats_body,
        grid=(_B, _S // _TS1),
        in_specs=[
            pl.BlockSpec((1, _TS1, _K, _D), lambda b, j: (b, j, 0, 0)),
            pl.BlockSpec((1, _TS1, _K, 16), lambda b, j: (b, j, 0, 0)),
        ],
        out_specs=pl.BlockSpec((1, 8, 128), lambda b, j: (b, 0, 0)),
        out_shape=jax.ShapeDtypeStruct((_B, 8, 128), jnp.float32),
    )(g1, g2)


_TS2 = 64


def _final_body(g1_ref, g2_ref, sp_ref, st_ref, a_ref, b_ref, out_ref):
    std = st_ref[0, 0, 2]
    denom = std + 1e-5
    a1 = a_ref[0, 0, :, 0:_D]
    b1 = b_ref[0, 0, :, 0:_D]
    a2 = a_ref[0, 0, :, _D:_C2]
    b2 = b_ref[0, 0, :, _D:_C2]
    g1 = g1_ref[...]
    m1 = jnp.mean(g1, axis=2, keepdims=True)
    n1 = a1[None, None] * ((g1 - m1) / denom) + b1[None, None]
    g2 = g2_ref[0, :, :, 0:2]
    m2 = jnp.mean(g2, axis=1, keepdims=True)
    n2 = a2[None, None] * ((g2 - m2) / denom)[None] + b2[None, None]
    rep = jnp.broadcast_to(sp_ref[0][:, None, :], (_TS2, _K, _D))[None]
    out_ref[...] = jnp.concatenate([n1, n2, rep], axis=-1)


def _final(g1, g2, spts, st, alpha, beta):
    a4 = alpha.reshape(1, 1, 1, _C2)
    b4 = beta.reshape(1, 1, 1, _C2)
    return pl.pallas_call(
        _final_body,
        grid=(_B, _S // _TS2),
        in_specs=[
            pl.BlockSpec((1, _TS2, _K, _D), lambda b, j: (b, j, 0, 0)),
            pl.BlockSpec((1, _TS2, _K, 16), lambda b, j: (b, j, 0, 0)),
            pl.BlockSpec((1, _TS2, _D), lambda b, j: (b, j, 0)),
            pl.BlockSpec((1, 8, 128), lambda b, j: (b, 0, 0)),
            pl.BlockSpec((1, 1, 1, _C2), lambda b, j: (0, 0, 0, 0)),
            pl.BlockSpec((1, 1, 1, _C2), lambda b, j: (0, 0, 0, 0)),
        ],
        out_specs=pl.BlockSpec((1, _TS2, _K, _COUT), lambda b, j: (b, j, 0, 0)),
        out_shape=jax.ShapeDtypeStruct((_B, _S, _K, _COUT), jnp.float32),
    )(g1, g2, spts, st, a4, b4)


def kernel(xy, points, affine_alpha, affine_beta):
    sidx, nx, ny = _fps(xy)
    new_xy = jnp.stack([nx.reshape(_B, _S), ny.reshape(_B, _S)], axis=-1)
    idx = _knn(xy, nx, ny)
    boff = (jnp.arange(_B, dtype=jnp.int32) * _N).reshape(_B, 1, 1)
    sgidx = sidx.reshape(_B, _S) + boff[:, :, 0]
    gpts, gxy, spts = _gather(points, xy, idx, sgidx)
    g1 = gpts.reshape(_B, _S, _K, _D)
    g2 = gxy.reshape(_B, _S, _K, 16)
    sp = spts.reshape(_B, _S, _D)
    st = _stats(g1, g2)
    new_points = _final(g1, g2, sp, st, affine_alpha, affine_beta)
    return new_xy, new_points


# speedup vs baseline: 5.8739x; 1.4294x over previous
import functools

import jax
import jax.numpy as jnp
from jax import lax
from jax.experimental import pallas as pl
from jax.experimental.pallas import tpu as pltpu
from jax.experimental.pallas import tpu_sc as plsc

_B, _N, _D = 8, 4096, 128
_S = 1024
_K = 32
_C2 = _D + 2
_COUT = 2 * _D + 2
_SR = _S // 128
_NR = _N // 128


def _fps_body(xy_ref, sidx_ref, nx_ref, ny_ref):
    xv = xy_ref[:, 0]
    yv = xy_ref[:, 1]
    iota_n = (lax.broadcasted_iota(jnp.int32, (1, _NR, 128), 1) * 128
              + lax.broadcasted_iota(jnp.int32, (1, _NR, 128), 2))
    iota_s = (lax.broadcasted_iota(jnp.int32, (1, _SR, 128), 1) * 128
              + lax.broadcasted_iota(jnp.int32, (1, _SR, 128), 2))

    def body(i, st):
        dist, f, sacc, cxa, cya = st
        m = iota_n == f
        cx = jnp.sum(jnp.where(m, xv, 0.0), axis=(1, 2), keepdims=True)
        cy = jnp.sum(jnp.where(m, yv, 0.0), axis=(1, 2), keepdims=True)
        pm = iota_s == i
        sacc = jnp.where(pm, f, sacc)
        cxa = jnp.where(pm, cx, cxa)
        cya = jnp.where(pm, cy, cya)
        d = (xv - cx) ** 2 + (yv - cy) ** 2
        dist = jnp.minimum(dist, d)
        gmax = jnp.max(dist, axis=(1, 2), keepdims=True)
        f = jnp.min(jnp.where(dist == gmax, iota_n, 2 ** 30),
                    axis=(1, 2), keepdims=True)
        return dist, f, sacc, cxa, cya

    dist0 = jnp.full((_B, _NR, 128), 1e10, dtype=jnp.float32)
    f0 = jnp.zeros((_B, 1, 1), dtype=jnp.int32)
    sacc0 = jnp.zeros((_B, _SR, 128), dtype=jnp.int32)
    z0 = jnp.zeros((_B, _SR, 128), dtype=jnp.float32)
    _, _, sacc, cxa, cya = lax.fori_loop(0, _S, body,
                                         (dist0, f0, sacc0, z0, z0))
    sidx_ref[...] = sacc
    nx_ref[...] = cxa
    ny_ref[...] = cya


def _fps(xy):
    xyt = xy.transpose(0, 2, 1).reshape(_B, 2, _NR, 128)
    sidx, nx, ny = pl.pallas_call(
        _fps_body,
        out_shape=(
            jax.ShapeDtypeStruct((_B, _SR, 128), jnp.int32),
            jax.ShapeDtypeStruct((_B, _SR, 128), jnp.float32),
            jax.ShapeDtypeStruct((_B, _SR, 128), jnp.float32),
        ),
    )(xyt)
    return sidx, nx, ny


_RT = 32
_DEPTH = 6


def _knn_body(xyt_ref, nqx_ref, nqy_ref, idx_ref):
    b = pl.program_id(0)
    x = xyt_ref[0, 0]
    y = xyt_ref[0, 1]
    sx = nqx_ref[0]
    sy = nqy_ref[0]
    q = jnp.concatenate([sx, sy], axis=1)
    xyt2 = jnp.concatenate([x, y], axis=0)
    t = jax.lax.dot_general(q, xyt2, (((1,), (0,)), ((), ())),
                            preferred_element_type=jnp.float32)
    d0 = -2.0 * t + (sx * sx + sy * sy) + (x * x + y * y)

    big = jnp.int32(2 ** 30)
    inf = jnp.float32(jnp.inf)
    lane = lax.broadcasted_iota(jnp.int32, (_RT, 128), 1)
    n3 = (lax.broadcasted_iota(jnp.int32, (1, _NR, 128), 1) * 128
          + lax.broadcasted_iota(jnp.int32, (1, _NR, 128), 2))

    rem = d0.reshape(_RT, _NR, 128)
    sv, sn = [], []
    for lev in range(_DEPTH):
        mv = rem[:, 0]
        mn = lane
        for j in range(1, _NR):
            c = rem[:, j]
            lt = c < mv
            mn = jnp.where(lt, lane + j * 128, mn)
            mv = jnp.where(lt, c, mv)
        sv.append(mv)
        sn.append(mn)
        if lev < _DEPTH - 1:
            rem = jnp.where(n3 == mn[:, None, :], inf, rem)

    cnt = jnp.zeros((_RT, 128), jnp.int32)
    hv, hn = sv[0], sn[0]
    cols = []
    for _ in range(_K):
        gmin = jnp.min(hv, axis=1, keepdims=True)
        nsel = jnp.min(jnp.where(hv == gmin, hn, big), axis=1, keepdims=True)
        cols.append(nsel)
        consumed = hn == nsel
        cnt = cnt + jnp.where(consumed, 1, 0)
        nv = jnp.full((_RT, 128), inf, jnp.float32)
        nn = jnp.full((_RT, 128), big, jnp.int32)
        for lev in range(1, _DEPTH):
            m = cnt == lev
            nv = jnp.where(m, sv[lev], nv)
            nn = jnp.where(m, sn[lev], nn)
        hv = jnp.where(consumed, nv, hv)
        hn = jnp.where(consumed, nn, hn)
    fast = jnp.concatenate(cols, axis=1)

    def _naive():
        d = d0
        iota = lax.broadcasted_iota(jnp.int32, (_RT, _N), 1)
        cs = []
        for _ in range(_K):
            g = jnp.min(d, axis=1, keepdims=True)
            ni = jnp.min(jnp.where(d == g, iota, big), axis=1, keepdims=True)
            cs.append(ni)
            d = jnp.where(iota == ni, inf, d)
        return jnp.concatenate(cs, axis=1)

    res = lax.cond(jnp.max(cnt) >= _DEPTH, _naive, lambda: fast)
    idx_ref[0] = res + b * _N


def _knn(xy, nx, ny):
    xyt = xy.transpose(0, 2, 1).reshape(_B, 2, 1, _N)
    nqx = nx.reshape(_B, _S, 1)
    nqy = ny.reshape(_B, _S, 1)
    idx = pl.pallas_call(
        _knn_body,
        grid=(_B, _S // _RT),
        in_specs=[
            pl.BlockSpec((1, 2, 1, _N), lambda b, j: (b, 0, 0, 0)),
            pl.BlockSpec((1, _RT, 1), lambda b, j: (b, j, 0)),
            pl.BlockSpec((1, _RT, 1), lambda b, j: (b, j, 0)),
        ],
        out_specs=pl.BlockSpec((1, _RT, _K), lambda b, j: (b, j, 0)),
        out_shape=jax.ShapeDtypeStruct((_B, _S, _K), jnp.int32),
    )(xyt, nqx, nqy)
    return idx


_NC, _NS = 2, 16
_NW = _NC * _NS
_GCH = 256
_GPW = (_B * _S * _K) // _NW
_SPW = (_B * _S) // _NW


def _gather_body(pts_hbm, xyp_hbm, gidx_hbm, sgidx_hbm,
                 gpts_hbm, gxy_hbm, spts_hbm,
                 idx_v, rows_v, xyrows_v, sidx_v, srows_v, sem):
    wid = lax.axis_index("s") * _NC + lax.axis_index("c")
    base0 = wid * _GPW

    def chunk(i, carry):
        base = pl.multiple_of(base0 + i * _GCH, _GCH)
        pltpu.sync_copy(gidx_hbm.at[pl.ds(base, _GCH)], idx_v)
        pltpu.async_copy(pts_hbm.at[idx_v], rows_v, sem).wait()
        pltpu.sync_copy(rows_v, gpts_hbm.at[pl.ds(base, _GCH)])
        pltpu.async_copy(xyp_hbm.at[idx_v], xyrows_v, sem).wait()
        pltpu.sync_copy(xyrows_v, gxy_hbm.at[pl.ds(base, _GCH)])
        return carry

    lax.fori_loop(0, _GPW // _GCH, chunk, 0)

    sbase = pl.multiple_of(wid * _SPW, _SPW)
    pltpu.sync_copy(sgidx_hbm.at[pl.ds(sbase, _SPW)], sidx_v)
    pltpu.async_copy(pts_hbm.at[sidx_v], srows_v, sem).wait()
    pltpu.sync_copy(srows_v, spts_hbm.at[pl.ds(sbase, _SPW)])


def _gather(points, xy, gidx, sgidx):
    pts_tab = points.reshape(_B * _N, _D)
    xy_tab = jnp.pad(xy.reshape(_B * _N, 2), ((0, 0), (0, 14)))
    mesh = plsc.VectorSubcoreMesh(core_axis_name="c", subcore_axis_name="s")
    run = pl.kernel(
        _gather_body,
        mesh=mesh,
        compiler_params=pltpu.CompilerParams(use_tc_tiling_on_sc=False),
        out_type=(
            jax.ShapeDtypeStruct((_B * _S * _K, _D), jnp.float32),
            jax.ShapeDtypeStruct((_B * _S * _K, 16), jnp.float32),
            jax.ShapeDtypeStruct((_B * _S, _D), jnp.float32),
        ),
        scratch_types=[
            pltpu.VMEM((_GCH,), jnp.int32),
            pltpu.VMEM((_GCH, _D), jnp.float32),
            pltpu.VMEM((_GCH, 16), jnp.float32),
            pltpu.VMEM((_SPW,), jnp.int32),
            pltpu.VMEM((_SPW, _D), jnp.float32),
            pltpu.SemaphoreType.DMA,
        ],
    )
    return run(pts_tab, xy_tab, gidx.reshape(-1), sgidx.reshape(-1))


_TS1 = 128
_M_TOT = _S * _K * _C2


def _stats_body(g1_ref, g2_ref, st_ref):
    j = pl.program_id(1)
    g1 = g1_ref[...]
    m1 = jnp.mean(g1, axis=2, keepdims=True)
    y1 = g1 - m1
    g2 = g2_ref[0, :, :, 0:2]
    m2 = jnp.mean(g2, axis=1, keepdims=True)
    y2 = g2 - m2
    s = jnp.sum(y1) + jnp.sum(y2)
    q = jnp.sum(y1 * y1) + jnp.sum(y2 * y2)
    lane = lax.broadcasted_iota(jnp.int32, (1, 8, 128), 2)
    sub = lax.broadcasted_iota(jnp.int32, (1, 8, 128), 1)
    oh0 = jnp.where((lane == 0) & (sub == 0), 1.0, 0.0)
    oh1 = jnp.where((lane == 1) & (sub == 0), 1.0, 0.0)
    oh2 = (lane == 2) & (sub == 0)

    @pl.when(j == 0)
    def _():
        st_ref[...] = jnp.zeros((1, 8, 128), jnp.float32)

    st_ref[...] += s * oh0 + q * oh1

    @pl.when(j == (_S // _TS1) - 1)
    def _():
        st = st_ref[...]
        s_tot = jnp.sum(st * oh0)
        q_tot = jnp.sum(st * oh1)
        var = (q_tot - s_tot * s_tot / _M_TOT) / (_M_TOT - 1)
        std = jnp.sqrt(var)
        st_ref[...] = jnp.where(oh2, std, st)


def _stats(g1, g2):
    return pl.pallas_call(
        _stats_body,
        grid=(_B, _S // _TS1),
        in_specs=[
            pl.BlockSpec((1, _TS1, _K, _D), lambda b, j: (b, j, 0, 0)),
            pl.BlockSpec((1, _TS1, _K, 16), lambda b, j: (b, j, 0, 0)),
        ],
        out_specs=pl.BlockSpec((1, 8, 128), lambda b, j: (b, 0, 0)),
        out_shape=jax.ShapeDtypeStruct((_B, 8, 128), jnp.float32),
    )(g1, g2)


_TS2 = 64


def _final_body(g1_ref, g2_ref, sp_ref, st_ref, a_ref, b_ref, out_ref):
    std = st_ref[0, 0, 2]
    denom = std + 1e-5
    a1 = a_ref[0, 0, :, 0:_D]
    b1 = b_ref[0, 0, :, 0:_D]
    a2 = a_ref[0, 0, :, _D:_C2]
    b2 = b_ref[0, 0, :, _D:_C2]
    g1 = g1_ref[...]
    m1 = jnp.mean(g1, axis=2, keepdims=True)
    n1 = a1[None, None] * ((g1 - m1) / denom) + b1[None, None]
    g2 = g2_ref[0, :, :, 0:2]
    m2 = jnp.mean(g2, axis=1, keepdims=True)
    n2 = a2[None, None] * ((g2 - m2) / denom)[None] + b2[None, None]
    rep = jnp.broadcast_to(sp_ref[0][:, None, :], (_TS2, _K, _D))[None]
    out_ref[...] = jnp.concatenate([n1, n2, rep], axis=-1)


def _final(g1, g2, spts, st, alpha, beta):
    a4 = alpha.reshape(1, 1, 1, _C2)
    b4 = beta.reshape(1, 1, 1, _C2)
    return pl.pallas_call(
        _final_body,
        grid=(_B, _S // _TS2),
        in_specs=[
            pl.BlockSpec((1, _TS2, _K, _D), lambda b, j: (b, j, 0, 0)),
            pl.BlockSpec((1, _TS2, _K, 16), lambda b, j: (b, j, 0, 0)),
            pl.BlockSpec((1, _TS2, _D), lambda b, j: (b, j, 0)),
            pl.BlockSpec((1, 8, 128), lambda b, j: (b, 0, 0)),
            pl.BlockSpec((1, 1, 1, _C2), lambda b, j: (0, 0, 0, 0)),
            pl.BlockSpec((1, 1, 1, _C2), lambda b, j: (0, 0, 0, 0)),
        ],
        out_specs=pl.BlockSpec((1, _TS2, _K, _COUT), lambda b, j: (b, j, 0, 0)),
        out_shape=jax.ShapeDtypeStruct((_B, _S, _K, _COUT), jnp.float32),
    )(g1, g2, spts, st, a4, b4)


def kernel(xy, points, affine_alpha, affine_beta):
    sidx, nx, ny = _fps(xy)
    new_xy = jnp.stack([nx.reshape(_B, _S), ny.reshape(_B, _S)], axis=-1)
    idx = _knn(xy, nx, ny)
    boff = (jnp.arange(_B, dtype=jnp.int32) * _N).reshape(_B, 1, 1)
    sgidx = sidx.reshape(_B, _S) + boff[:, :, 0]
    gpts, gxy, spts = _gather(points, xy, idx, sgidx)
    g1 = gpts.reshape(_B, _S, _K, _D)
    g2 = gxy.reshape(_B, _S, _K, 16)
    sp = spts.reshape(_B, _S, _D)
    st = _stats(g1, g2)
    new_points = _final(g1, g2, sp, st, affine_alpha, affine_beta)
    return new_xy, new_points


# speedup vs baseline: 6.8637x; 1.1685x over previous
import functools

import jax
import jax.numpy as jnp
from jax import lax
from jax.experimental import pallas as pl
from jax.experimental.pallas import tpu as pltpu
from jax.experimental.pallas import tpu_sc as plsc

_B, _N, _D = 8, 4096, 128
_S = 1024
_K = 32
_C2 = _D + 2
_COUT = 2 * _D + 2
_SR = _S // 128
_NR = _N // 128


def _fps_body(xy_ref, sidx_ref, nx_ref, ny_ref):
    xv = xy_ref[:, 0]
    yv = xy_ref[:, 1]
    iota_n = (lax.broadcasted_iota(jnp.int32, (1, _NR, 128), 1) * 128
              + lax.broadcasted_iota(jnp.int32, (1, _NR, 128), 2))
    iota_s = (lax.broadcasted_iota(jnp.int32, (1, _SR, 128), 1) * 128
              + lax.broadcasted_iota(jnp.int32, (1, _SR, 128), 2))

    def body(i, st):
        dist, f, sacc, cxa, cya = st
        m = iota_n == f
        cx = jnp.sum(jnp.where(m, xv, 0.0), axis=(1, 2), keepdims=True)
        cy = jnp.sum(jnp.where(m, yv, 0.0), axis=(1, 2), keepdims=True)
        pm = iota_s == i
        sacc = jnp.where(pm, f, sacc)
        cxa = jnp.where(pm, cx, cxa)
        cya = jnp.where(pm, cy, cya)
        d = (xv - cx) ** 2 + (yv - cy) ** 2
        dist = jnp.minimum(dist, d)
        gmax = jnp.max(dist, axis=(1, 2), keepdims=True)
        f = jnp.min(jnp.where(dist == gmax, iota_n, 2 ** 30),
                    axis=(1, 2), keepdims=True)
        return dist, f, sacc, cxa, cya

    dist0 = jnp.full((_B, _NR, 128), 1e10, dtype=jnp.float32)
    f0 = jnp.zeros((_B, 1, 1), dtype=jnp.int32)
    sacc0 = jnp.zeros((_B, _SR, 128), dtype=jnp.int32)
    z0 = jnp.zeros((_B, _SR, 128), dtype=jnp.float32)
    _, _, sacc, cxa, cya = lax.fori_loop(0, _S, body,
                                         (dist0, f0, sacc0, z0, z0))
    sidx_ref[...] = sacc
    nx_ref[...] = cxa
    ny_ref[...] = cya


def _fps(xy):
    xyt = xy.transpose(0, 2, 1).reshape(_B, 2, _NR, 128)
    sidx, nx, ny = pl.pallas_call(
        _fps_body,
        out_shape=(
            jax.ShapeDtypeStruct((_B, _SR, 128), jnp.int32),
            jax.ShapeDtypeStruct((_B, _SR, 128), jnp.float32),
            jax.ShapeDtypeStruct((_B, _SR, 128), jnp.float32),
        ),
    )(xyt)
    return sidx, nx, ny


_RT = 64
_DEPTH = 6


def _knn_body(xyt_ref, nqx_ref, nqy_ref, idx_ref):
    b = pl.program_id(0)
    x = xyt_ref[0, 0]
    y = xyt_ref[0, 1]
    sx = nqx_ref[0]
    sy = nqy_ref[0]
    q = jnp.concatenate([sx, sy], axis=1)
    xyt2 = jnp.concatenate([x, y], axis=0)
    t = jax.lax.dot_general(q, xyt2, (((1,), (0,)), ((), ())),
                            preferred_element_type=jnp.float32)
    d0 = -2.0 * t + (sx * sx + sy * sy) + (x * x + y * y)

    big = jnp.int32(2 ** 30)
    inf = jnp.float32(jnp.inf)
    lane = lax.broadcasted_iota(jnp.int32, (_RT, 128), 1)
    n3 = (lax.broadcasted_iota(jnp.int32, (1, _NR, 128), 1) * 128
          + lax.broadcasted_iota(jnp.int32, (1, _NR, 128), 2))

    rem = d0.reshape(_RT, _NR, 128)
    sv, sn = [], []
    for lev in range(_DEPTH):
        mv = rem[:, 0]
        mn = lane
        for j in range(1, _NR):
            c = rem[:, j]
            lt = c < mv
            mn = jnp.where(lt, lane + j * 128, mn)
            mv = jnp.where(lt, c, mv)
        sv.append(mv)
        sn.append(mn)
        if lev < _DEPTH - 1:
            rem = jnp.where(n3 == mn[:, None, :], inf, rem)

    cnt = jnp.zeros((_RT, 128), jnp.int32)
    hv, hn = sv[0], sn[0]
    cols = []
    for _ in range(_K):
        gmin = jnp.min(hv, axis=1, keepdims=True)
        nsel = jnp.min(jnp.where(hv == gmin, hn, big), axis=1, keepdims=True)
        cols.append(nsel)
        consumed = hn == nsel
        cnt = cnt + jnp.where(consumed, 1, 0)
        nv = jnp.full((_RT, 128), inf, jnp.float32)
        nn = jnp.full((_RT, 128), big, jnp.int32)
        for lev in range(1, _DEPTH):
            m = cnt == lev
            nv = jnp.where(m, sv[lev], nv)
            nn = jnp.where(m, sn[lev], nn)
        hv = jnp.where(consumed, nv, hv)
        hn = jnp.where(consumed, nn, hn)
    fast = jnp.concatenate(cols, axis=1)

    def _naive():
        d = d0
        iota = lax.broadcasted_iota(jnp.int32, (_RT, _N), 1)
        cs = []
        for _ in range(_K):
            g = jnp.min(d, axis=1, keepdims=True)
            ni = jnp.min(jnp.where(d == g, iota, big), axis=1, keepdims=True)
            cs.append(ni)
            d = jnp.where(iota == ni, inf, d)
        return jnp.concatenate(cs, axis=1)

    res = lax.cond(jnp.max(cnt) >= _DEPTH, _naive, lambda: fast)
    idx_ref[0] = res + b * _N


def _knn(xy, nx, ny):
    xyt = xy.transpose(0, 2, 1).reshape(_B, 2, 1, _N)
    nqx = nx.reshape(_B, _S, 1)
    nqy = ny.reshape(_B, _S, 1)
    idx = pl.pallas_call(
        _knn_body,
        grid=(_B, _S // _RT),
        in_specs=[
            pl.BlockSpec((1, 2, 1, _N), lambda b, j: (b, 0, 0, 0)),
            pl.BlockSpec((1, _RT, 1), lambda b, j: (b, j, 0)),
            pl.BlockSpec((1, _RT, 1), lambda b, j: (b, j, 0)),
        ],
        out_specs=pl.BlockSpec((1, _RT, _K), lambda b, j: (b, j, 0)),
        out_shape=jax.ShapeDtypeStruct((_B, _S, _K), jnp.int32),
    )(xyt, nqx, nqy)
    return idx


_NC, _NS = 2, 16
_NW = _NC * _NS
_GCH = 256
_GPW = (_B * _S * _K) // _NW
_SPW = (_B * _S) // _NW


def _gather_body(pts_hbm, xyp_hbm, gidx_hbm, sgidx_hbm,
                 gpts_hbm, gxy_hbm, spts_hbm,
                 idx_v, rows_v, xyrows_v, sidx_v, srows_v, sem):
    wid = lax.axis_index("s") * _NC + lax.axis_index("c")
    base0 = wid * _GPW

    def chunk(i, carry):
        base = pl.multiple_of(base0 + i * _GCH, _GCH)
        pltpu.sync_copy(gidx_hbm.at[pl.ds(base, _GCH)], idx_v)
        pltpu.async_copy(pts_hbm.at[idx_v], rows_v, sem).wait()
        pltpu.sync_copy(rows_v, gpts_hbm.at[pl.ds(base, _GCH)])
        pltpu.async_copy(xyp_hbm.at[idx_v], xyrows_v, sem).wait()
        pltpu.sync_copy(xyrows_v, gxy_hbm.at[pl.ds(base, _GCH)])
        return carry

    lax.fori_loop(0, _GPW // _GCH, chunk, 0)

    sbase = pl.multiple_of(wid * _SPW, _SPW)
    pltpu.sync_copy(sgidx_hbm.at[pl.ds(sbase, _SPW)], sidx_v)
    pltpu.async_copy(pts_hbm.at[sidx_v], srows_v, sem).wait()
    pltpu.sync_copy(srows_v, spts_hbm.at[pl.ds(sbase, _SPW)])


def _gather(points, xy, gidx, sgidx):
    pts_tab = points.reshape(_B * _N, _D)
    xy_tab = jnp.pad(xy.reshape(_B * _N, 2), ((0, 0), (0, 14)))
    mesh = plsc.VectorSubcoreMesh(core_axis_name="c", subcore_axis_name="s")
    run = pl.kernel(
        _gather_body,
        mesh=mesh,
        compiler_params=pltpu.CompilerParams(use_tc_tiling_on_sc=False),
        out_type=(
            jax.ShapeDtypeStruct((_B * _S * _K, _D), jnp.float32),
            jax.ShapeDtypeStruct((_B * _S * _K, 16), jnp.float32),
            jax.ShapeDtypeStruct((_B * _S, _D), jnp.float32),
        ),
        scratch_types=[
            pltpu.VMEM((_GCH,), jnp.int32),
            pltpu.VMEM((_GCH, _D), jnp.float32),
            pltpu.VMEM((_GCH, 16), jnp.float32),
            pltpu.VMEM((_SPW,), jnp.int32),
            pltpu.VMEM((_SPW, _D), jnp.float32),
            pltpu.SemaphoreType.DMA,
        ],
    )
    return run(pts_tab, xy_tab, gidx.reshape(-1), sgidx.reshape(-1))


_TS1 = 128
_M_TOT = _S * _K * _C2


def _stats_body(g1_ref, g2_ref, st_ref):
    j = pl.program_id(1)
    g1 = g1_ref[...]
    m1 = jnp.mean(g1, axis=2, keepdims=True)
    y1 = g1 - m1
    g2 = g2_ref[0, :, :, 0:2]
    m2 = jnp.mean(g2, axis=1, keepdims=True)
    y2 = g2 - m2
    s = jnp.sum(y1) + jnp.sum(y2)
    q = jnp.sum(y1 * y1) + jnp.sum(y2 * y2)
    lane = lax.broadcasted_iota(jnp.int32, (1, 8, 128), 2)
    sub = lax.broadcasted_iota(jnp.int32, (1, 8, 128), 1)
    oh0 = jnp.where((lane == 0) & (sub == 0), 1.0, 0.0)
    oh1 = jnp.where((lane == 1) & (sub == 0), 1.0, 0.0)
    oh2 = (lane == 2) & (sub == 0)

    @pl.when(j == 0)
    def _():
        st_ref[...] = jnp.zeros((1, 8, 128), jnp.float32)

    st_ref[...] += s * oh0 + q * oh1

    @pl.when(j == (_S // _TS1) - 1)
    def _():
        st = st_ref[...]
        s_tot = jnp.sum(st * oh0)
        q_tot = jnp.sum(st * oh1)
        var = (q_tot - s_tot * s_tot / _M_TOT) / (_M_TOT - 1)
        std = jnp.sqrt(var)
        st_ref[...] = jnp.where(oh2, std, st)


def _stats(g1, g2):
    return pl.pallas_call(
        _stats_body,
        grid=(_B, _S // _TS1),
        in_specs=[
            pl.BlockSpec((1, _TS1, _K, _D), lambda b, j: (b, j, 0, 0)),
            pl.BlockSpec((1, _TS1, _K, 16), lambda b, j: (b, j, 0, 0)),
        ],
        out_specs=pl.BlockSpec((1, 8, 128), lambda b, j: (b, 0, 0)),
        out_shape=jax.ShapeDtypeStruct((_B, 8, 128), jnp.float32),
    )(g1, g2)


_TS2 = 64


def _final_body(g1_ref, g2_ref, sp_ref, st_ref, a_ref, b_ref, out_ref):
    std = st_ref[0, 0, 2]
    denom = std + 1e-5
    a1 = a_ref[0, 0, :, 0:_D]
    b1 = b_ref[0, 0, :, 0:_D]
    a2 = a_ref[0, 0, :, _D:_C2]
    b2 = b_ref[0, 0, :, _D:_C2]
    g1 = g1_ref[...]
    m1 = jnp.mean(g1, axis=2, keepdims=True)
    n1 = a1[None, None] * ((g1 - m1) / denom) + b1[None, None]
    g2 = g2_ref[0, :, :, 0:2]
    m2 = jnp.mean(g2, axis=1, keepdims=True)
    n2 = a2[None, None] * ((g2 - m2) / denom)[None] + b2[None, None]
    rep = jnp.broadcast_to(sp_ref[0][:, None, :], (_TS2, _K, _D))[None]
    out_ref[...] = jnp.concatenate([n1, n2, rep], axis=-1)


def _final(g1, g2, spts, st, alpha, beta):
    a4 = alpha.reshape(1, 1, 1, _C2)
    b4 = beta.reshape(1, 1, 1, _C2)
    return pl.pallas_call(
        _final_body,
        grid=(_B, _S // _TS2),
        in_specs=[
            pl.BlockSpec((1, _TS2, _K, _D), lambda b, j: (b, j, 0, 0)),
            pl.BlockSpec((1, _TS2, _K, 16), lambda b, j: (b, j, 0, 0)),
            pl.BlockSpec((1, _TS2, _D), lambda b, j: (b, j, 0)),
            pl.BlockSpec((1, 8, 128), lambda b, j: (b, 0, 0)),
            pl.BlockSpec((1, 1, 1, _C2), lambda b, j: (0, 0, 0, 0)),
            pl.BlockSpec((1, 1, 1, _C2), lambda b, j: (0, 0, 0, 0)),
        ],
        out_specs=pl.BlockSpec((1, _TS2, _K, _COUT), lambda b, j: (b, j, 0, 0)),
        out_shape=jax.ShapeDtypeStruct((_B, _S, _K, _COUT), jnp.float32),
    )(g1, g2, spts, st, a4, b4)


def kernel(xy, points, affine_alpha, affine_beta):
    sidx, nx, ny = _fps(xy)
    new_xy = jnp.stack([nx.reshape(_B, _S), ny.reshape(_B, _S)], axis=-1)
    idx = _knn(xy, nx, ny)
    boff = (jnp.arange(_B, dtype=jnp.int32) * _N).reshape(_B, 1, 1)
    sgidx = sidx.reshape(_B, _S) + boff[:, :, 0]
    gpts, gxy, spts = _gather(points, xy, idx, sgidx)
    g1 = gpts.reshape(_B, _S, _K, _D)
    g2 = gxy.reshape(_B, _S, _K, 16)
    sp = spts.reshape(_B, _S, _D)
    st = _stats(g1, g2)
    new_points = _final(g1, g2, sp, st, affine_alpha, affine_beta)
    return new_xy, new_points
